# baseline (speedup 1.0000x reference)
import jax
import jax.numpy as jnp
from jax import lax
from jax.experimental import pallas as pl
from jax.experimental.pallas import tpu as pltpu

HIDDEN = 4096
NUM_EXPERTS = 64
NUM_TOKENS = 32768

BT = 1024
NBUF = 3

_DN = (((1,), (1,)), ((), ()))


def _body(x_hbm, w_ref, b_ref, o_hbm):
    def inner(x_blk, o_blk):
        o_blk[...] = (
            lax.dot_general(x_blk[...], w_ref[...], _DN,
                            preferred_element_type=jnp.float32)
            + b_ref[...]
        )

    pltpu.emit_pipeline(
        inner,
        grid=(NUM_TOKENS // BT,),
        in_specs=[
            pl.BlockSpec((BT, HIDDEN), lambda i: (i, 0),
                         pipeline_mode=pl.Buffered(buffer_count=NBUF,
                                                   use_lookahead=True)),
        ],
        out_specs=[
            pl.BlockSpec((BT, NUM_EXPERTS), lambda i: (i, 0)),
        ],
    )(x_hbm, o_hbm)


def kernel(x, W, b):
    b2 = b.reshape(1, NUM_EXPERTS)
    return pl.pallas_call(
        _body,
        in_specs=[
            pl.BlockSpec(memory_space=pltpu.HBM),
            pl.BlockSpec((NUM_EXPERTS, HIDDEN), lambda: (0, 0)),
            pl.BlockSpec((1, NUM_EXPERTS), lambda: (0, 0)),
        ],
        out_specs=pl.BlockSpec(memory_space=pltpu.HBM),
        out_shape=jax.ShapeDtypeStruct((NUM_TOKENS, NUM_EXPERTS), jnp.float32),
        compiler_params=pltpu.CompilerParams(
            vmem_limit_bytes=63 * 1024 * 1024,
        ),
    )(x, W, b2)


# speedup vs baseline: 1.0047x; 1.0047x over previous
import jax
import jax.numpy as jnp
from jax import lax
from jax.experimental import pallas as pl
from jax.experimental.pallas import tpu as pltpu

HIDDEN = 4096
NUM_EXPERTS = 64
NUM_TOKENS = 32768

BT = 1024

_DN = (((1,), (1,)), ((), ()))


def _router_block(x_ref, w_ref, b_ref, o_ref):
    i = pl.program_id(0)
    o_ref[pl.ds(i * BT, BT), :] = (
        lax.dot_general(x_ref[...], w_ref[...], _DN,
                        preferred_element_type=jnp.float32)
        + b_ref[...]
    )


def kernel(x, W, b):
    b2 = b.reshape(1, NUM_EXPERTS)
    grid = (NUM_TOKENS // BT,)
    return pl.pallas_call(
        _router_block,
        grid=grid,
        in_specs=[
            pl.BlockSpec((BT, HIDDEN), lambda i: (i, 0)),
            pl.BlockSpec((NUM_EXPERTS, HIDDEN), lambda i: (0, 0)),
            pl.BlockSpec((1, NUM_EXPERTS), lambda i: (0, 0)),
        ],
        out_specs=pl.BlockSpec((NUM_TOKENS, NUM_EXPERTS), lambda i: (0, 0)),
        out_shape=jax.ShapeDtypeStruct((NUM_TOKENS, NUM_EXPERTS), jnp.float32),
        compiler_params=pltpu.CompilerParams(
            dimension_semantics=("arbitrary",),
            vmem_limit_bytes=63 * 1024 * 1024,
            skip_device_barrier=True,
        ),
    )(x, W, b2)


# speedup vs baseline: 1.0146x; 1.0098x over previous
import jax
import jax.numpy as jnp
from jax import lax
from jax.experimental import pallas as pl
from jax.experimental.pallas import tpu as pltpu

HIDDEN = 4096
NUM_EXPERTS = 64
NUM_TOKENS = 32768

BT = 512
K = 4
HC = HIDDEN // K

_DN = (((1,), (1,)), ((), ()))


def _router_block(*refs):
    x_refs = refs[:K]
    w_refs = refs[K:2 * K]
    b_ref = refs[2 * K]
    o_ref = refs[2 * K + 1]
    acc = lax.dot_general(x_refs[0][...], w_refs[0][...], _DN,
                          preferred_element_type=jnp.float32)
    for k in range(1, K):
        acc += lax.dot_general(x_refs[k][...], w_refs[k][...], _DN,
                               preferred_element_type=jnp.float32)
    o_ref[...] = acc + b_ref[...]


def kernel(x, W, b):
    b2 = b.reshape(1, NUM_EXPERTS)
    grid = (NUM_TOKENS // BT,)
    x_specs = [
        pl.BlockSpec((BT, HC), lambda i, k=k: (i, k)) for k in range(K)
    ]
    w_specs = [
        pl.BlockSpec((NUM_EXPERTS, HC), lambda i, k=k: (0, k)) for k in range(K)
    ]
    return pl.pallas_call(
        _router_block,
        grid=grid,
        in_specs=x_specs + w_specs + [
            pl.BlockSpec((1, NUM_EXPERTS), lambda i: (0, 0)),
        ],
        out_specs=pl.BlockSpec((BT, NUM_EXPERTS), lambda i: (i, 0)),
        out_shape=jax.ShapeDtypeStruct((NUM_TOKENS, NUM_EXPERTS), jnp.float32),
        compiler_params=pltpu.CompilerParams(
            dimension_semantics=("parallel",),
        ),
    )(*([x] * K + [W] * K + [b2]))
